# baseline (speedup 1.0000x reference)
import jax
import jax.numpy as jnp
from jax import lax
from jax.experimental import pallas as pl
from jax.experimental.pallas import tpu as pltpu
from jax.experimental.pallas import tpu_sc as plsc

N = 8388608
NC = 2
NS = 16
NW = NC * NS
PER_W = N // NW
CHUNK = 8192
NCHUNK = PER_W // CHUNK
GROUPS = CHUNK // 64


def _body(x_hbm, xq_hbm, pb_hbm, x_v0, x_v1, xq_v0, xq_v1, pb_v0, pb_v1,
          isem, oqsem, opsem):
    x_v = (x_v0, x_v1)
    xq_v = (xq_v0, xq_v1)
    pb_v = (pb_v0, pb_v1)
    wid = lax.axis_index("s") * NC + lax.axis_index("c")
    base = wid * PER_W
    strided = [lax.iota(jnp.int32, 16) * 4 + c for c in range(4)]

    def in_copy(t):
        b = t & 1
        return pltpu.make_async_copy(
            x_hbm.at[pl.ds(base + t * CHUNK, CHUNK)], x_v[b], isem.at[b]
        )

    def out_copies(t):
        b = t & 1
        return (
            pltpu.make_async_copy(
                xq_v[b], xq_hbm.at[pl.ds(base + t * CHUNK, CHUNK)],
                oqsem.at[b],
            ),
            pltpu.make_async_copy(
                pb_v[b], pb_hbm.at[pl.ds(base + t * CHUNK, CHUNK)],
                opsem.at[b],
            ),
        )

    in_copy(0).start()
    for t in range(NCHUNK):
        b = t & 1
        if t + 1 < NCHUNK:
            in_copy(t + 1).start()
        in_copy(t).wait()
        if t >= 2:
            for cp in out_copies(t - 2):
                cp.wait()

        @plsc.parallel_loop(0, GROUPS, unroll=2)
        def group(g):
            gbase = g * 64
            packed = jnp.zeros((16,), jnp.int32)
            for c in range(4):
                ii = gbase + strided[c]
                x = plsc.load_gather(x_v[b], [ii])
                i = (
                    jnp.where(x > -1.0, 1, 0)
                    + jnp.where(x > 0.0, 1, 0)
                    + jnp.where(x > 1.0, 1, 0)
                )
                q = i.astype(jnp.float32) - 1.5
                plsc.store_scatter(xq_v[b], [ii], q)
                packed = packed | (i << (8 * c)) if c else i
            pb_v[b][pl.ds(gbase, 64)] = plsc.bitcast(packed, jnp.uint8)

        for cp in out_copies(t):
            cp.start()
    for t in (NCHUNK - 2, NCHUNK - 1):
        for cp in out_copies(t):
            cp.wait()


_sc_quantize = pl.kernel(
    _body,
    out_type=[
        jax.ShapeDtypeStruct((N,), jnp.float32),
        jax.ShapeDtypeStruct((N,), jnp.uint8),
    ],
    mesh=plsc.VectorSubcoreMesh(
        core_axis_name="c", subcore_axis_name="s", num_cores=NC, num_subcores=NS
    ),
    scratch_types=[
        pltpu.VMEM((CHUNK,), jnp.float32),
        pltpu.VMEM((CHUNK,), jnp.float32),
        pltpu.VMEM((CHUNK,), jnp.float32),
        pltpu.VMEM((CHUNK,), jnp.float32),
        pltpu.VMEM((CHUNK,), jnp.uint8),
        pltpu.VMEM((CHUNK,), jnp.uint8),
        pltpu.SemaphoreType.DMA((2,)),
        pltpu.SemaphoreType.DMA((2,)),
        pltpu.SemaphoreType.DMA((2,)),
    ],
    compiler_params=pltpu.CompilerParams(
        needs_layout_passes=False, use_tc_tiling_on_sc=False
    ),
)


@jax.jit
def kernel(X):
    xq, idx = _sc_quantize(X.reshape(-1))
    return (xq.reshape(-1, 1), idx)


# speedup vs baseline: 1.0405x; 1.0405x over previous
import jax
import jax.numpy as jnp
from jax import lax
from jax.experimental import pallas as pl
from jax.experimental.pallas import tpu as pltpu
from jax.experimental.pallas import tpu_sc as plsc

N = 8388608
NC = 2
NS = 16
NW = NC * NS
PER_W = N // NW
CHUNK = 16384
NCHUNK = PER_W // CHUNK
GROUPS = CHUNK // 64


def _body(x_hbm, xq_hbm, pb_hbm, x_v0, x_v1, xq_v0, xq_v1, pb_v0, pb_v1,
          isem, oqsem, opsem):
    x_v = (x_v0, x_v1)
    xq_v = (xq_v0, xq_v1)
    pb_v = (pb_v0, pb_v1)
    wid = lax.axis_index("s") * NC + lax.axis_index("c")
    base = wid * PER_W
    strided = [lax.iota(jnp.int32, 16) * 4 + c for c in range(4)]

    def in_copy(t):
        b = t & 1
        return pltpu.make_async_copy(
            x_hbm.at[pl.ds(base + t * CHUNK, CHUNK)], x_v[b], isem.at[b]
        )

    def out_copies(t):
        b = t & 1
        return (
            pltpu.make_async_copy(
                xq_v[b], xq_hbm.at[pl.ds(base + t * CHUNK, CHUNK)],
                oqsem.at[b],
            ),
            pltpu.make_async_copy(
                pb_v[b], pb_hbm.at[pl.ds(base + t * CHUNK, CHUNK)],
                opsem.at[b],
            ),
        )

    in_copy(0).start()
    for t in range(NCHUNK):
        b = t & 1
        if t + 1 < NCHUNK:
            in_copy(t + 1).start()
        in_copy(t).wait()
        if t >= 2:
            for cp in out_copies(t - 2):
                cp.wait()

        @plsc.parallel_loop(0, GROUPS, unroll=2)
        def group(g):
            gbase = g * 64
            packed = jnp.zeros((16,), jnp.int32)
            for c in range(4):
                ii = gbase + strided[c]
                x = plsc.load_gather(x_v[b], [ii])
                i = (
                    jnp.where(x > -1.0, 1, 0)
                    + jnp.where(x > 0.0, 1, 0)
                    + jnp.where(x > 1.0, 1, 0)
                )
                q = i.astype(jnp.float32) - 1.5
                plsc.store_scatter(xq_v[b], [ii], q)
                packed = packed | (i << (8 * c)) if c else i
            pb_v[b][pl.ds(gbase, 64)] = plsc.bitcast(packed, jnp.uint8)

        for cp in out_copies(t):
            cp.start()
    for t in (NCHUNK - 2, NCHUNK - 1):
        for cp in out_copies(t):
            cp.wait()


_sc_quantize = pl.kernel(
    _body,
    out_type=[
        jax.ShapeDtypeStruct((N,), jnp.float32),
        jax.ShapeDtypeStruct((N,), jnp.uint8),
    ],
    mesh=plsc.VectorSubcoreMesh(
        core_axis_name="c", subcore_axis_name="s", num_cores=NC, num_subcores=NS
    ),
    scratch_types=[
        pltpu.VMEM((CHUNK,), jnp.float32),
        pltpu.VMEM((CHUNK,), jnp.float32),
        pltpu.VMEM((CHUNK,), jnp.float32),
        pltpu.VMEM((CHUNK,), jnp.float32),
        pltpu.VMEM((CHUNK,), jnp.uint8),
        pltpu.VMEM((CHUNK,), jnp.uint8),
        pltpu.SemaphoreType.DMA((2,)),
        pltpu.SemaphoreType.DMA((2,)),
        pltpu.SemaphoreType.DMA((2,)),
    ],
    compiler_params=pltpu.CompilerParams(
        needs_layout_passes=False, use_tc_tiling_on_sc=False
    ),
)


@jax.jit
def kernel(X):
    xq, idx = _sc_quantize(X.reshape(-1))
    return (xq.reshape(-1, 1), idx)
